# baseline (speedup 1.0000x reference)
import functools

import jax
import jax.numpy as jnp
from jax import lax
from jax.experimental import pallas as pl
from jax.experimental.pallas import tpu as pltpu
from jax.experimental.pallas import tpu_sc as plsc

D_MODEL = 128
HEIGHT = 512
CENTER = 256 * 256 - 1
NUM_WORKERS = 32
H_PER_W = HEIGHT // NUM_WORKERS
BAND_ROWS = HEIGHT + H_PER_W
JQ = 128
NQ = HEIGHT // JQ
LANES = 16
D_GROUPS = D_MODEL // LANES


def _sc_body(w_hbm, x_hbm, o_hbm, band_v, x_v, buf_v):
    wid = lax.axis_index("s") * 2 + lax.axis_index("c")
    h0 = wid * H_PER_W
    start = CENTER - h0 - (H_PER_W - 1)
    pltpu.sync_copy(w_hbm.at[pl.ds(start, BAND_ROWS)], band_v)
    pltpu.sync_copy(x_hbm, x_v)

    def t_body(t, carry):
        def q_body(q, carry_q):
            def j_body(jc, carry_j):
                jbase = q * JQ + jc * LANES
                xv = x_v[pl.ds(jbase, LANES)]
                for i in range(LANES):
                    j = jbase + i
                    jj = jc * LANES + i
                    r = j + (H_PER_W - 1) - t
                    splat = jnp.full((LANES,), xv[i], jnp.float32)
                    for c in range(D_GROUPS):
                        sl = pl.ds(c * LANES, LANES)
                        buf_v[jj, sl] = band_v[r, sl] + splat
                return carry_j

            lax.fori_loop(0, JQ // LANES, j_body, 0)
            pltpu.sync_copy(buf_v, o_hbm.at[h0 + t, pl.ds(q * JQ, JQ)])
            return carry_q

        lax.fori_loop(0, NQ, q_body, 0)
        return carry

    lax.fori_loop(0, H_PER_W, t_body, 0)


_sc_kernel = functools.partial(
    pl.kernel,
    out_type=jax.ShapeDtypeStruct((HEIGHT, HEIGHT, D_MODEL), jnp.float32),
    mesh=plsc.VectorSubcoreMesh(core_axis_name="c", subcore_axis_name="s"),
    scratch_types=[
        pltpu.VMEM((BAND_ROWS, D_MODEL), jnp.float32),
        pltpu.VMEM((HEIGHT,), jnp.float32),
        pltpu.VMEM((JQ, D_MODEL), jnp.float32),
    ],
)(_sc_body)


def kernel(x, weight):
    xr = x.reshape(HEIGHT)
    out = _sc_kernel(weight, xr)
    return out.reshape(1, HEIGHT, 1, HEIGHT, D_MODEL)


# speedup vs baseline: 1.7596x; 1.7596x over previous
import functools

import jax
import jax.numpy as jnp
from jax import lax
from jax.experimental import pallas as pl
from jax.experimental.pallas import tpu as pltpu
from jax.experimental.pallas import tpu_sc as plsc

D_MODEL = 128
HEIGHT = 512
CENTER = 256 * 256 - 1
NUM_WORKERS = 32
H_PER_W = HEIGHT // NUM_WORKERS
BAND_ROWS = HEIGHT + H_PER_W
JQ = 128
NQ = HEIGHT // JQ
LANES = 16
D_GROUPS = D_MODEL // LANES


def _sc_body(w_hbm, x_hbm, o_hbm, band_v, x_v, buf0_v, buf1_v, sem0, sem1):
    wid = lax.axis_index("s") * 2 + lax.axis_index("c")
    h0 = wid * H_PER_W
    start = CENTER - h0 - (H_PER_W - 1)
    pltpu.sync_copy(w_hbm.at[pl.ds(start, BAND_ROWS)], band_v)
    pltpu.sync_copy(x_hbm, x_v)

    bufs = (buf0_v, buf1_v)
    sems = (sem0, sem1)

    def compute_tile(t, q, buf):
        @plsc.parallel_loop(0, JQ // LANES)
        def j_body(jc):
            jbase = q * JQ + jc * LANES
            xv = x_v[pl.ds(jbase, LANES)]
            for i in range(LANES):
                jj = jc * LANES + i
                r = jbase + i + (H_PER_W - 1) - t
                splat = jnp.full((LANES,), xv[i], jnp.float32)
                for c in range(D_GROUPS):
                    sl = pl.ds(c * LANES, LANES)
                    buf[jj, sl] = band_v[r, sl] + splat

    def pair_body(tq2, carry):
        for b in range(2):
            tq = tq2 * 2 + b
            t = tq // NQ
            q = tq % NQ

            @pl.when(tq2 >= 1)
            def _wait():
                pltpu.make_async_copy(
                    bufs[b], o_hbm.at[h0, pl.ds(0, JQ)], sems[b]
                ).wait()

            compute_tile(t, q, bufs[b])
            pltpu.async_copy(
                bufs[b], o_hbm.at[h0 + t, pl.ds(q * JQ, JQ)], sems[b]
            )
        return carry

    lax.fori_loop(0, H_PER_W * NQ // 2, pair_body, 0)
    for b in range(2):
        pltpu.make_async_copy(bufs[b], o_hbm.at[h0, pl.ds(0, JQ)], sems[b]).wait()


_sc_kernel = functools.partial(
    pl.kernel,
    out_type=jax.ShapeDtypeStruct((HEIGHT, HEIGHT, D_MODEL), jnp.float32),
    mesh=plsc.VectorSubcoreMesh(core_axis_name="c", subcore_axis_name="s"),
    scratch_types=[
        pltpu.VMEM((BAND_ROWS, D_MODEL), jnp.float32),
        pltpu.VMEM((HEIGHT,), jnp.float32),
        pltpu.VMEM((JQ, D_MODEL), jnp.float32),
        pltpu.VMEM((JQ, D_MODEL), jnp.float32),
        pltpu.SemaphoreType.DMA,
        pltpu.SemaphoreType.DMA,
    ],
)(_sc_body)


def kernel(x, weight):
    xr = x.reshape(HEIGHT)
    out = _sc_kernel(weight, xr)
    return out.reshape(1, HEIGHT, 1, HEIGHT, D_MODEL)


# speedup vs baseline: 4.0266x; 2.2883x over previous
import functools

import jax
import jax.numpy as jnp
from jax import lax
from jax.experimental import pallas as pl
from jax.experimental.pallas import tpu as pltpu
from jax.experimental.pallas import tpu_sc as plsc

D_MODEL = 128
HEIGHT = 512
CENTER = 256 * 256 - 1
BAND_START = CENTER - (HEIGHT - 1)
BAND_ROWS = 1024
NUM_WORKERS = 32
H_PER_W = HEIGHT // NUM_WORKERS
JQ = 128
NQ = HEIGHT // JQ
NBUF = 4
LANES = 16
D_GROUPS = D_MODEL // LANES


def _sc_body(w_hbm, x_hbm, o_hbm, band_sh, x_v, b0, b1, b2, b3,
             si0, si1, si2, si3, so0, so1, so2, so3):
    cid = lax.axis_index("c")
    sid = lax.axis_index("s")
    wid = sid * 2 + cid
    h0 = wid * H_PER_W

    bufs = (b0, b1, b2, b3)
    sin = (si0, si1, si2, si3)
    sout = (so0, so1, so2, so3)

    @pl.when(sid == 0)
    def _load_band():
        pltpu.sync_copy(w_hbm.at[pl.ds(BAND_START, BAND_ROWS)], band_sh)

    pltpu.sync_copy(x_hbm, x_v)
    plsc.subcore_barrier()

    def start_in(t, b):
        g0 = (HEIGHT - 1) + b * JQ - h0 - t
        pltpu.async_copy(band_sh.at[pl.ds(g0, JQ)], bufs[b], sin[b])

    def wait_in(b):
        pltpu.make_async_copy(band_sh.at[pl.ds(0, JQ)], bufs[b], sin[b]).wait()

    def start_out(t, b):
        pltpu.async_copy(bufs[b], o_hbm.at[h0 + t, pl.ds(b * JQ, JQ)], sout[b])

    def wait_out(b):
        pltpu.make_async_copy(bufs[b], o_hbm.at[h0, pl.ds(0, JQ)], sout[b]).wait()

    def add_x(b):
        buf = bufs[b]

        @plsc.parallel_loop(0, JQ // LANES)
        def _jc_body(jc):
            jbase = b * JQ + jc * LANES
            xv = x_v[pl.ds(jbase, LANES)]
            for i in range(LANES):
                jj = jc * LANES + i
                splat = jnp.full((LANES,), xv[i], jnp.float32)
                for c in range(D_GROUPS):
                    plsc.addupdate(buf.at[jj, pl.ds(c * LANES, LANES)], splat)

    start_in(0, 0)
    start_in(0, 1)

    def g_body(g, carry):
        for b in range(NBUF):
            i2 = g * NQ + b + 2
            b2 = (b + 2) % NBUF
            if b < 2:
                @pl.when(g >= 1)
                def _drain():
                    wait_out(b2)

                start_in(i2 // NQ, b2)
            else:
                @pl.when(g <= H_PER_W - 2)
                def _rearm():
                    wait_out(b2)
                    start_in(i2 // NQ, b2)

            wait_in(b)
            add_x(b)
            start_out(g, b)
        return carry

    lax.fori_loop(0, H_PER_W, g_body, 0)
    for b in range(NBUF):
        wait_out(b)


_sc_kernel = functools.partial(
    pl.kernel,
    out_type=jax.ShapeDtypeStruct((HEIGHT, HEIGHT, D_MODEL), jnp.float32),
    mesh=plsc.VectorSubcoreMesh(core_axis_name="c", subcore_axis_name="s"),
    scratch_types=[
        pltpu.VMEM_SHARED((BAND_ROWS, D_MODEL), jnp.float32),
        pltpu.VMEM((HEIGHT,), jnp.float32),
        pltpu.VMEM((JQ, D_MODEL), jnp.float32),
        pltpu.VMEM((JQ, D_MODEL), jnp.float32),
        pltpu.VMEM((JQ, D_MODEL), jnp.float32),
        pltpu.VMEM((JQ, D_MODEL), jnp.float32),
        pltpu.SemaphoreType.DMA,
        pltpu.SemaphoreType.DMA,
        pltpu.SemaphoreType.DMA,
        pltpu.SemaphoreType.DMA,
        pltpu.SemaphoreType.DMA,
        pltpu.SemaphoreType.DMA,
        pltpu.SemaphoreType.DMA,
        pltpu.SemaphoreType.DMA,
    ],
)(_sc_body)


def kernel(x, weight):
    xr = x.reshape(HEIGHT)
    out = _sc_kernel(weight, xr)
    return out.reshape(1, HEIGHT, 1, HEIGHT, D_MODEL)
